# Initial kernel scaffold; baseline (speedup 1.0000x reference)
import functools

import jax
import jax.numpy as jnp
from jax import lax
from jax.experimental import pallas as pl
from jax.experimental.pallas import tpu as pltpu
from jax.experimental.pallas import tpu_sc as plsc

LORA_SCALING = 2.0

NC = 2
NS = 16
NW = NC * NS

IDXV = 128
CHUNK = 1024
NSTREAM = CHUNK // IDXV


def _sc_gather_body(nchunk, x_hbm, w_hbm, a_hbm, base_hbm, arows_hbm,
                    idx_v, bufw, bufa, sem_i, sem_w, sem_a):
    wid = lax.axis_index("s") * NC + lax.axis_index("c")
    row0 = wid * (nchunk * NSTREAM)

    def chunk_body(c, carry):
        r = row0 + c * NSTREAM
        pltpu.async_copy(x_hbm.at[pl.ds(r, NSTREAM)], idx_v, sem_i).wait()
        descs = []
        for j in range(NSTREAM):
            descs.append(pltpu.async_copy(
                w_hbm.at[idx_v.at[j]], bufw.at[pl.ds(j * IDXV, IDXV)], sem_w))
            descs.append(pltpu.async_copy(
                a_hbm.at[idx_v.at[j]], bufa.at[pl.ds(j * IDXV, IDXV)], sem_a))
        for d in descs:
            d.wait()
        off = r * IDXV
        pltpu.sync_copy(bufw, base_hbm.at[pl.ds(off, CHUNK)])
        pltpu.sync_copy(bufa, arows_hbm.at[pl.ds(off, CHUNK)])
        return carry

    lax.fori_loop(0, nchunk, chunk_body, 0)


def _sc_gather(x2d, weight, lora_A):
    n = x2d.shape[0] * x2d.shape[1]
    nchunk = n // (NW * CHUNK)
    d = weight.shape[1]
    r = lora_A.shape[1]
    mesh = plsc.VectorSubcoreMesh(core_axis_name="c", subcore_axis_name="s",
                                  num_cores=NC, num_subcores=NS)
    kern = pl.kernel(
        functools.partial(_sc_gather_body, nchunk),
        out_type=(
            jax.ShapeDtypeStruct((n, d), jnp.float32),
            jax.ShapeDtypeStruct((n, r), jnp.float32),
        ),
        mesh=mesh,
        scratch_types=[
            pltpu.VMEM((NSTREAM, IDXV), jnp.int32),
            pltpu.VMEM((CHUNK, d), jnp.float32),
            pltpu.VMEM((CHUNK, r), jnp.float32),
            pltpu.SemaphoreType.DMA,
            pltpu.SemaphoreType.DMA,
            pltpu.SemaphoreType.DMA,
        ],
    )
    return kern(x2d, weight, lora_A)


def _tc_combine_body(base_ref, arows_ref, b_ref, out_ref):
    delta = lax.dot_general(arows_ref[...], b_ref[...],
                            (((1,), (0,)), ((), ())),
                            preferred_element_type=jnp.float32)
    out_ref[...] = base_ref[...] + delta * LORA_SCALING


def _tc_combine(base, arows, lora_B):
    n, d = base.shape
    r = arows.shape[1]
    blk = 4096
    return pl.pallas_call(
        _tc_combine_body,
        grid=(n // blk,),
        in_specs=[
            pl.BlockSpec((blk, d), lambda i: (i, 0)),
            pl.BlockSpec((blk, r), lambda i: (i, 0)),
            pl.BlockSpec((r, d), lambda i: (0, 0)),
        ],
        out_specs=pl.BlockSpec((blk, d), lambda i: (i, 0)),
        out_shape=jax.ShapeDtypeStruct((n, d), jnp.float32),
    )(base, arows, lora_B)


def kernel(x, weight, lora_A, lora_B):
    b, h = x.shape
    n = b * h
    x2d = x.reshape(n // IDXV, IDXV).astype(jnp.int32)
    base, arows = _sc_gather(x2d, weight, lora_A)
    out = _tc_combine(base, arows, lora_B)
    return out.reshape(b, h, weight.shape[1])


# speedup vs baseline: 6.8179x; 6.8179x over previous
import functools

import jax
import jax.numpy as jnp
from jax import lax
from jax.experimental import pallas as pl
from jax.experimental.pallas import tpu as pltpu
from jax.experimental.pallas import tpu_sc as plsc

LORA_SCALING = 2.0

NC = 2
NS = 16
NW = NC * NS

IDXV = 128
CHUNK = 1024
NSTREAM = CHUNK // IDXV


def _sc_gather_body(nchunk, x_hbm, w_hbm, a_hbm, base_hbm, arows_hbm,
                    idx_v, bufw, bufa, sem_i, sem_w, sem_a):
    wid = lax.axis_index("s") * NC + lax.axis_index("c")
    row0 = wid * (nchunk * NSTREAM)

    def chunk_body(c, carry):
        r = row0 + c * NSTREAM
        pltpu.async_copy(x_hbm.at[pl.ds(r, NSTREAM)], idx_v, sem_i).wait()
        descs = []
        for j in range(NSTREAM):
            descs.append(pltpu.async_copy(
                w_hbm.at[idx_v.at[j]], bufw.at[pl.ds(j * IDXV, IDXV)], sem_w))
            descs.append(pltpu.async_copy(
                a_hbm.at[idx_v.at[j]], bufa.at[pl.ds(j * IDXV, IDXV)], sem_a))
        for d in descs:
            d.wait()
        off = r * IDXV
        pltpu.sync_copy(bufw, base_hbm.at[pl.ds(off, CHUNK)])
        pltpu.sync_copy(bufa, arows_hbm.at[pl.ds(off, CHUNK)])
        return carry

    lax.fori_loop(0, nchunk, chunk_body, 0)


def _sc_gather(x2d, weight, lora_A):
    n = x2d.shape[0] * x2d.shape[1]
    nchunk = n // (NW * CHUNK)
    d = weight.shape[1]
    r = lora_A.shape[1]
    mesh = plsc.VectorSubcoreMesh(core_axis_name="c", subcore_axis_name="s",
                                  num_cores=NC, num_subcores=NS)
    kern = pl.kernel(
        functools.partial(_sc_gather_body, nchunk),
        out_type=(
            jax.ShapeDtypeStruct((n, d), jnp.float32),
            jax.ShapeDtypeStruct((n, r), jnp.float32),
        ),
        mesh=mesh,
        scratch_types=[
            pltpu.VMEM((NSTREAM, IDXV), jnp.int32),
            pltpu.VMEM((CHUNK, d), jnp.float32),
            pltpu.VMEM((CHUNK, r), jnp.float32),
            pltpu.SemaphoreType.DMA,
            pltpu.SemaphoreType.DMA,
            pltpu.SemaphoreType.DMA,
        ],
        compiler_params=pltpu.CompilerParams(use_tc_tiling_on_sc=False),
    )
    return kern(x2d, weight, lora_A)


def _tc_combine_body(base_ref, arows_ref, b_ref, out_ref):
    delta = lax.dot_general(arows_ref[...], b_ref[...],
                            (((1,), (0,)), ((), ())),
                            preferred_element_type=jnp.float32)
    out_ref[...] = base_ref[...] + delta * LORA_SCALING


def _tc_combine(base, arows, lora_B):
    n, d = base.shape
    r = arows.shape[1]
    blk = 4096
    return pl.pallas_call(
        _tc_combine_body,
        grid=(n // blk,),
        in_specs=[
            pl.BlockSpec((blk, d), lambda i: (i, 0)),
            pl.BlockSpec((blk, r), lambda i: (i, 0)),
            pl.BlockSpec((r, d), lambda i: (0, 0)),
        ],
        out_specs=pl.BlockSpec((blk, d), lambda i: (i, 0)),
        out_shape=jax.ShapeDtypeStruct((n, d), jnp.float32),
    )(base, arows, lora_B)


def kernel(x, weight, lora_A, lora_B):
    b, h = x.shape
    n = b * h
    x2d = x.reshape(n // IDXV, IDXV).astype(jnp.int32)
    base, arows = _sc_gather(x2d, weight, lora_A)
    out = _tc_combine(base, arows, lora_B)
    return out.reshape(b, h, weight.shape[1])
